# Initial kernel scaffold; baseline (speedup 1.0000x reference)
import jax
import jax.numpy as jnp
from jax.experimental import pallas as pl


def kernel(x, edge_index, edge_attr, batch, W0, b0, g0, be0, W1, b1, g1, be1, W2, b2, W3, b3, fcW, fcb):
    raise NotImplementedError("write your pallas kernel here")


# speedup vs baseline: 12.5587x; 12.5587x over previous
import functools

import jax
import jax.numpy as jnp
from jax import lax
from jax.experimental import pallas as pl
from jax.experimental.pallas import tpu as pltpu
from jax.experimental.pallas import tpu_sc as plsc

N = 10000
E = 320000
D = 128
H = 64
G = 64

NC = 2
NS = 16
NW = NC * NS
K = 128
NCH = 80
EP = NW * NCH * K
NP = 10240
RPT = NP // NS
RPW = NP // NW

_mesh = plsc.VectorSubcoreMesh(core_axis_name="c", subcore_axis_name="s")


def _deg_body(coli, ew, out, coli_v, ew_v, zb, acc):
    c = lax.axis_index("c")
    s = lax.axis_index("s")
    wid = c * NS + s
    pltpu.sync_copy(coli.at[wid], coli_v)
    pltpu.sync_copy(ew.at[wid], ew_v)
    zeros = jnp.zeros((16,), jnp.float32)
    for j in range(RPT // 16):
        zb[pl.ds(j * 16, 16)] = zeros
    base = pl.multiple_of(s * RPT, RPT)
    pltpu.sync_copy(zb, acc.at[pl.ds(base, RPT)])
    plsc.subcore_barrier()

    @pl.loop(0, NCH)
    def _edge_chunks(i):
        pltpu.sync_copy(ew_v.at[i], acc.at[coli_v.at[i]], add=True)

    plsc.subcore_barrier()
    pltpu.sync_copy(acc.at[pl.ds(base, RPT)], out.at[c, pl.ds(base, RPT)])


_deg_kernel = functools.partial(
    pl.kernel,
    out_type=jax.ShapeDtypeStruct((NC, NP), jnp.float32),
    mesh=_mesh,
    compiler_params=pltpu.CompilerParams(use_tc_tiling_on_sc=False),
    scratch_types=[
        pltpu.VMEM((NCH, K), jnp.int32),
        pltpu.VMEM((NCH, K), jnp.float32),
        pltpu.VMEM((RPT,), jnp.float32),
        pltpu.VMEM_SHARED((NP,), jnp.float32),
    ],
)(_deg_body)


def _mp_body(h, rowi, coli, ew, out, rowi_v, coli_v, ew_v, rows0, rows1,
             acc, sem0, sem1):
    c = lax.axis_index("c")
    s = lax.axis_index("s")
    wid = c * NS + s
    pltpu.sync_copy(rowi.at[wid], rowi_v)
    pltpu.sync_copy(coli.at[wid], coli_v)
    pltpu.sync_copy(ew.at[wid], ew_v)

    zeros = jnp.zeros((16,), jnp.float32)

    @pl.loop(0, K)
    def _zero_rows(k):
        for cc in range(4):
            rows0[k, pl.ds(cc * 16, 16)] = zeros

    base = pl.multiple_of(s * RPT, RPT)
    for t in range(RPT // K):
        pltpu.sync_copy(rows0, acc.at[pl.ds(base + t * K, K)])
    plsc.subcore_barrier()

    def scale_scatter(i, buf):
        @pl.loop(0, K // 16)
        def _scale(g):
            wv = ew_v[i, pl.ds(g * 16, 16)]
            for kk in range(16):
                w = wv[kk]
                k = g * 16 + kk
                for cc in range(4):
                    sl = pl.ds(cc * 16, 16)
                    buf[k, sl] = buf[k, sl] * w
        pltpu.sync_copy(buf, acc.at[coli_v.at[i]], add=True)

    pltpu.async_copy(h.at[rowi_v.at[0]], rows0, sem0)

    @pl.loop(0, NCH // 2)
    def _pairs(j):
        i0 = j * 2
        pltpu.async_copy(h.at[rowi_v.at[i0 + 1]], rows1, sem1)
        pltpu.make_async_copy(h.at[rowi_v.at[i0]], rows0, sem0).wait()
        scale_scatter(i0, rows0)

        @pl.when(j < NCH // 2 - 1)
        def _prefetch():
            pltpu.async_copy(h.at[rowi_v.at[i0 + 2]], rows0, sem0)

        pltpu.make_async_copy(h.at[rowi_v.at[i0 + 1]], rows1, sem1).wait()
        scale_scatter(i0 + 1, rows1)

    plsc.subcore_barrier()
    pltpu.sync_copy(acc.at[pl.ds(base, RPT)], out.at[c, pl.ds(base, RPT)])


_mp_kernel = functools.partial(
    pl.kernel,
    out_type=jax.ShapeDtypeStruct((NC, NP, H), jnp.float32),
    mesh=_mesh,
    compiler_params=pltpu.CompilerParams(use_tc_tiling_on_sc=False),
    scratch_types=[
        pltpu.VMEM((NCH, K), jnp.int32),
        pltpu.VMEM((NCH, K), jnp.int32),
        pltpu.VMEM((NCH, K), jnp.float32),
        pltpu.VMEM((K, H), jnp.float32),
        pltpu.VMEM((K, H), jnp.float32),
        pltpu.VMEM_SHARED((NP, H), jnp.float32),
        pltpu.SemaphoreType.DMA,
        pltpu.SemaphoreType.DMA,
    ],
)(_mp_body)


def _max_body(h3p, batchp, out, h3_v, b_v, acc_v):
    c = lax.axis_index("c")
    s = lax.axis_index("s")
    wid = c * NS + s
    base = pl.multiple_of(wid * RPW, RPW)
    pltpu.sync_copy(h3p.at[pl.ds(base, RPW)], h3_v)
    pltpu.sync_copy(batchp.at[pl.ds(base, RPW)], b_v)
    neg = jnp.full((16,), -jnp.inf, jnp.float32)

    @pl.loop(0, G)
    def _init(g):
        for cc in range(4):
            acc_v[g, pl.ds(cc * 16, 16)] = neg

    ngroups = jnp.minimum(RPW, N - wid * RPW) // 16

    @pl.loop(0, ngroups)
    def _rows(g):
        bg = b_v[pl.ds(g * 16, 16)]
        for kk in range(16):
            gi = bg[kk]
            r = g * 16 + kk
            for cc in range(4):
                sl = pl.ds(cc * 16, 16)
                acc_v[gi, sl] = jnp.maximum(acc_v[gi, sl], h3_v[r, sl])

    pltpu.sync_copy(acc_v, out.at[wid])


_max_kernel = functools.partial(
    pl.kernel,
    out_type=jax.ShapeDtypeStruct((NW, G, H), jnp.float32),
    mesh=_mesh,
    compiler_params=pltpu.CompilerParams(use_tc_tiling_on_sc=False),
    scratch_types=[
        pltpu.VMEM((RPW, H), jnp.float32),
        pltpu.VMEM((RPW,), jnp.int32),
        pltpu.VMEM((G, H), jnp.float32),
    ],
)(_max_body)


def _tc_prep_body(degP, x, W0, h0p_ref, dis_ref):
    deg = degP[0, :N] + degP[1, :N] + 1.0
    dis = jnp.where(deg > 0, lax.rsqrt(deg), 0.0)[:, None]
    u = jnp.dot(x[...], W0[...], preferred_element_type=jnp.float32)
    h0p_ref[...] = dis * u
    dis_ref[...] = dis


_tc_prep = pl.pallas_call(
    _tc_prep_body,
    out_shape=[
        jax.ShapeDtypeStruct((N, H), jnp.float32),
        jax.ShapeDtypeStruct((N, 1), jnp.float32),
    ],
)


def _tc_combine_body(P, hp, dis, b, g, be, Wn, out_ref, *, bn):
    acc = P[0, :N] + P[1, :N] + hp[...]
    dis_v = dis[...]
    conv = dis_v * acc + b[...][None, :]
    if bn:
        m = jnp.mean(conv, axis=0, keepdims=True)
        v = jnp.mean((conv - m) ** 2, axis=0, keepdims=True)
        conv = (conv - m) * lax.rsqrt(v + 1e-5) * g[...][None, :] + be[...][None, :]
    hact = jnp.where(conv >= 0, conv, 0.01 * conv)
    u = jnp.dot(hact, Wn[...], preferred_element_type=jnp.float32)
    out_ref[...] = dis_v * u


def _make_combine(bn):
    return pl.pallas_call(
        functools.partial(_tc_combine_body, bn=bn),
        out_shape=jax.ShapeDtypeStruct((N, H), jnp.float32),
    )


_tc_combine_bn = _make_combine(True)
_tc_combine_nobn = _make_combine(False)


def _tc_final_a_body(P, hp, dis, b3, batch, h3p_ref, sm_ref, cnt_ref):
    acc = P[0, :N] + P[1, :N] + hp[...]
    conv = dis[...] * acc + b3[...][None, :]
    h3 = jnp.where(conv >= 0, conv, 0.01 * conv)
    h3p_ref[:N, :] = h3
    h3p_ref[N:, :] = jnp.zeros((NP - N, H), jnp.float32)
    bvec = batch[...]
    onehot = (bvec[:, None] == lax.broadcasted_iota(jnp.int32, (1, G), 1))
    M = onehot.astype(jnp.float32)
    sm_ref[...] = lax.dot_general(M, h3, (((0,), (0,)), ((), ())),
                                  preferred_element_type=jnp.float32)
    cnt_ref[...] = jnp.sum(M, axis=0)[:, None]


_tc_final_a = pl.pallas_call(
    _tc_final_a_body,
    out_shape=[
        jax.ShapeDtypeStruct((NP, H), jnp.float32),
        jax.ShapeDtypeStruct((G, H), jnp.float32),
        jax.ShapeDtypeStruct((G, 1), jnp.float32),
    ],
)


def _tc_final_b_body(maxP, sm, cnt, fcW, fcb, out_ref):
    mx = jnp.max(maxP[...], axis=0)
    mean = sm[...] / jnp.maximum(cnt[...], 1.0)
    pooled = jnp.concatenate([mx, mean], axis=1)
    out_ref[...] = (jnp.dot(pooled, fcW[...], preferred_element_type=jnp.float32)
                    + fcb[...][None, :])


_tc_final_b = pl.pallas_call(
    _tc_final_b_body,
    out_shape=jax.ShapeDtypeStruct((G, 1), jnp.float32),
)


def kernel(x, edge_index, edge_attr, batch, W0, b0, g0, be0, W1, b1, g1, be1,
           W2, b2, W3, b3, fcW, fcb):
    ipad = jnp.zeros((EP - E,), jnp.int32)
    row = jnp.concatenate([edge_index[0].astype(jnp.int32), ipad]
                          ).reshape(NW, NCH, K)
    col = jnp.concatenate([edge_index[1].astype(jnp.int32), ipad]
                          ).reshape(NW, NCH, K)
    ewr = jnp.concatenate([edge_attr, jnp.zeros((EP - E,), jnp.float32)]
                          ).reshape(NW, NCH, K)
    batchp = jnp.concatenate(
        [batch.astype(jnp.int32), jnp.zeros((NP - N,), jnp.int32)])

    degP = _deg_kernel(col, ewr)
    h0p, dis = _tc_prep(degP, x, W0)
    P0 = _mp_kernel(h0p, row, col, ewr)
    h1p = _tc_combine_bn(P0, h0p, dis, b0, g0, be0, W1)
    P1 = _mp_kernel(h1p, row, col, ewr)
    h2p = _tc_combine_bn(P1, h1p, dis, b1, g1, be1, W2)
    P2 = _mp_kernel(h2p, row, col, ewr)
    h3p_self = _tc_combine_nobn(P2, h2p, dis, b2, jnp.zeros_like(g0),
                                jnp.zeros_like(be0), W3)
    P3 = _mp_kernel(h3p_self, row, col, ewr)
    h3pad, sm, cnt = _tc_final_a(P3, h3p_self, dis, b3, batch.astype(jnp.int32))
    maxP = _max_kernel(h3pad, batchp)
    return _tc_final_b(maxP, sm, cnt, fcW, fcb)


# speedup vs baseline: 12.8993x; 1.0271x over previous
import functools

import jax
import jax.numpy as jnp
from jax import lax
from jax.experimental import pallas as pl
from jax.experimental.pallas import tpu as pltpu
from jax.experimental.pallas import tpu_sc as plsc

N = 10000
E = 320000
D = 128
H = 64
G = 64

NC = 2
NS = 16
NW = NC * NS
K = 128
NCH = 80
EP = NW * NCH * K
NP = 10240
RPT = NP // NS
RPW = NP // NW

_mesh = plsc.VectorSubcoreMesh(core_axis_name="c", subcore_axis_name="s")


def _deg_body(coli, ew, out, coli_v, ew_v, zb, acc):
    c = lax.axis_index("c")
    s = lax.axis_index("s")
    wid = c * NS + s
    pltpu.sync_copy(coli.at[wid], coli_v)
    pltpu.sync_copy(ew.at[wid], ew_v)
    zeros = jnp.zeros((16,), jnp.float32)
    for j in range(RPT // 16):
        zb[pl.ds(j * 16, 16)] = zeros
    base = pl.multiple_of(s * RPT, RPT)
    pltpu.sync_copy(zb, acc.at[pl.ds(base, RPT)])
    plsc.subcore_barrier()

    @pl.loop(0, NCH)
    def _edge_chunks(i):
        pltpu.sync_copy(ew_v.at[i], acc.at[coli_v.at[i]], add=True)

    plsc.subcore_barrier()
    pltpu.sync_copy(acc.at[pl.ds(base, RPT)], out.at[c, pl.ds(base, RPT)])


_deg_kernel = functools.partial(
    pl.kernel,
    out_type=jax.ShapeDtypeStruct((NC, NP), jnp.float32),
    mesh=_mesh,
    compiler_params=pltpu.CompilerParams(use_tc_tiling_on_sc=False),
    scratch_types=[
        pltpu.VMEM((NCH, K), jnp.int32),
        pltpu.VMEM((NCH, K), jnp.float32),
        pltpu.VMEM((RPT,), jnp.float32),
        pltpu.VMEM_SHARED((NP,), jnp.float32),
    ],
)(_deg_body)


def _mp_body(h, rowi, coli, ew, out, rowi_v, coli_v, ew_v, b0, b1, b2, b3,
             gs0, gs1, gs2, gs3, ss0, ss1, ss2, ss3, acc):
    bufs = (b0, b1, b2, b3)
    gsems = (gs0, gs1, gs2, gs3)
    ssems = (ss0, ss1, ss2, ss3)
    c = lax.axis_index("c")
    s = lax.axis_index("s")
    wid = c * NS + s
    pltpu.sync_copy(rowi.at[wid], rowi_v)
    pltpu.sync_copy(coli.at[wid], coli_v)
    pltpu.sync_copy(ew.at[wid], ew_v)

    zeros = jnp.zeros((16,), jnp.float32)

    @pl.loop(0, K)
    def _zero_rows(k):
        for cc in range(4):
            b0[k, pl.ds(cc * 16, 16)] = zeros

    base = pl.multiple_of(s * RPT, RPT)
    for t in range(RPT // K):
        pltpu.sync_copy(b0, acc.at[pl.ds(base + t * K, K)])
    plsc.subcore_barrier()

    def scale(i, buf):
        @pl.loop(0, K // 16)
        def _scale(g):
            wv = ew_v[i, pl.ds(g * 16, 16)]
            for kk in range(16):
                w = wv[kk]
                k = g * 16 + kk
                for cc in range(4):
                    sl = pl.ds(cc * 16, 16)
                    buf[k, sl] = buf[k, sl] * w

    for p in range(3):
        pltpu.async_copy(h.at[rowi_v.at[p]], bufs[p], gsems[p])

    @pl.loop(0, NCH // 4)
    def _quad(j):
        i0 = j * 4
        for phase in range(4):
            i = i0 + phase
            p = phase
            q = (phase + 3) % 4
            pltpu.make_async_copy(h.at[rowi_v.at[i]], bufs[p], gsems[p]).wait()
            scale(i, bufs[p])
            pltpu.async_copy(bufs[p], acc.at[coli_v.at[i]], ssems[p], add=True)
            if phase == 0:
                @pl.when(j > 0)
                def _wait_prev():
                    pltpu.make_async_copy(
                        bufs[3], acc.at[coli_v.at[i - 1]], ssems[3]).wait()
                pltpu.async_copy(h.at[rowi_v.at[i + 3]], bufs[q], gsems[q])
            else:
                pltpu.make_async_copy(
                    bufs[q], acc.at[coli_v.at[i - 1]], ssems[q]).wait()

                @pl.when(i + 3 < NCH)
                def _next_gather():
                    pltpu.async_copy(h.at[rowi_v.at[i + 3]], bufs[q], gsems[q])

    pltpu.make_async_copy(bufs[3], acc.at[coli_v.at[NCH - 1]], ssems[3]).wait()
    plsc.subcore_barrier()
    pltpu.sync_copy(acc.at[pl.ds(base, RPT)], out.at[c, pl.ds(base, RPT)])


_mp_kernel = functools.partial(
    pl.kernel,
    out_type=jax.ShapeDtypeStruct((NC, NP, H), jnp.float32),
    mesh=_mesh,
    compiler_params=pltpu.CompilerParams(use_tc_tiling_on_sc=False),
    scratch_types=[
        pltpu.VMEM((NCH, K), jnp.int32),
        pltpu.VMEM((NCH, K), jnp.int32),
        pltpu.VMEM((NCH, K), jnp.float32),
        pltpu.VMEM((K, H), jnp.float32),
        pltpu.VMEM((K, H), jnp.float32),
        pltpu.VMEM((K, H), jnp.float32),
        pltpu.VMEM((K, H), jnp.float32),
        pltpu.SemaphoreType.DMA,
        pltpu.SemaphoreType.DMA,
        pltpu.SemaphoreType.DMA,
        pltpu.SemaphoreType.DMA,
        pltpu.SemaphoreType.DMA,
        pltpu.SemaphoreType.DMA,
        pltpu.SemaphoreType.DMA,
        pltpu.SemaphoreType.DMA,
        pltpu.VMEM_SHARED((NP, H), jnp.float32),
    ],
)(_mp_body)


def _max_body(h3p, batchp, out, h3_v, b_v, acc_v):
    c = lax.axis_index("c")
    s = lax.axis_index("s")
    wid = c * NS + s
    base = pl.multiple_of(wid * RPW, RPW)
    pltpu.sync_copy(h3p.at[pl.ds(base, RPW)], h3_v)
    pltpu.sync_copy(batchp.at[pl.ds(base, RPW)], b_v)
    neg = jnp.full((16,), -jnp.inf, jnp.float32)

    @pl.loop(0, G)
    def _init(g):
        for cc in range(4):
            acc_v[g, pl.ds(cc * 16, 16)] = neg

    ngroups = jnp.minimum(RPW, N - wid * RPW) // 16

    @pl.loop(0, ngroups)
    def _rows(g):
        bg = b_v[pl.ds(g * 16, 16)]
        for kk in range(16):
            gi = bg[kk]
            r = g * 16 + kk
            for cc in range(4):
                sl = pl.ds(cc * 16, 16)
                acc_v[gi, sl] = jnp.maximum(acc_v[gi, sl], h3_v[r, sl])

    pltpu.sync_copy(acc_v, out.at[wid])


_max_kernel = functools.partial(
    pl.kernel,
    out_type=jax.ShapeDtypeStruct((NW, G, H), jnp.float32),
    mesh=_mesh,
    compiler_params=pltpu.CompilerParams(use_tc_tiling_on_sc=False),
    scratch_types=[
        pltpu.VMEM((RPW, H), jnp.float32),
        pltpu.VMEM((RPW,), jnp.int32),
        pltpu.VMEM((G, H), jnp.float32),
    ],
)(_max_body)


def _tc_prep_body(degP, x, W0, h0p_ref, dis_ref):
    deg = degP[0, :N] + degP[1, :N] + 1.0
    dis = jnp.where(deg > 0, lax.rsqrt(deg), 0.0)[:, None]
    u = jnp.dot(x[...], W0[...], preferred_element_type=jnp.float32)
    h0p_ref[...] = dis * u
    dis_ref[...] = dis


_tc_prep = pl.pallas_call(
    _tc_prep_body,
    out_shape=[
        jax.ShapeDtypeStruct((N, H), jnp.float32),
        jax.ShapeDtypeStruct((N, 1), jnp.float32),
    ],
)


def _tc_combine_body(P, hp, dis, b, g, be, Wn, out_ref, *, bn):
    acc = P[0, :N] + P[1, :N] + hp[...]
    dis_v = dis[...]
    conv = dis_v * acc + b[...][None, :]
    if bn:
        m = jnp.mean(conv, axis=0, keepdims=True)
        v = jnp.mean((conv - m) ** 2, axis=0, keepdims=True)
        conv = (conv - m) * lax.rsqrt(v + 1e-5) * g[...][None, :] + be[...][None, :]
    hact = jnp.where(conv >= 0, conv, 0.01 * conv)
    u = jnp.dot(hact, Wn[...], preferred_element_type=jnp.float32)
    out_ref[...] = dis_v * u


def _make_combine(bn):
    return pl.pallas_call(
        functools.partial(_tc_combine_body, bn=bn),
        out_shape=jax.ShapeDtypeStruct((N, H), jnp.float32),
    )


_tc_combine_bn = _make_combine(True)
_tc_combine_nobn = _make_combine(False)


def _tc_final_a_body(P, hp, dis, b3, batch, h3p_ref, sm_ref, cnt_ref):
    acc = P[0, :N] + P[1, :N] + hp[...]
    conv = dis[...] * acc + b3[...][None, :]
    h3 = jnp.where(conv >= 0, conv, 0.01 * conv)
    h3p_ref[:N, :] = h3
    h3p_ref[N:, :] = jnp.zeros((NP - N, H), jnp.float32)
    bvec = batch[...]
    onehot = (bvec[:, None] == lax.broadcasted_iota(jnp.int32, (1, G), 1))
    M = onehot.astype(jnp.float32)
    sm_ref[...] = lax.dot_general(M, h3, (((0,), (0,)), ((), ())),
                                  preferred_element_type=jnp.float32)
    cnt_ref[...] = jnp.sum(M, axis=0)[:, None]


_tc_final_a = pl.pallas_call(
    _tc_final_a_body,
    out_shape=[
        jax.ShapeDtypeStruct((NP, H), jnp.float32),
        jax.ShapeDtypeStruct((G, H), jnp.float32),
        jax.ShapeDtypeStruct((G, 1), jnp.float32),
    ],
)


def _tc_final_b_body(maxP, sm, cnt, fcW, fcb, out_ref):
    mx = jnp.max(maxP[...], axis=0)
    mean = sm[...] / jnp.maximum(cnt[...], 1.0)
    pooled = jnp.concatenate([mx, mean], axis=1)
    out_ref[...] = (jnp.dot(pooled, fcW[...], preferred_element_type=jnp.float32)
                    + fcb[...][None, :])


_tc_final_b = pl.pallas_call(
    _tc_final_b_body,
    out_shape=jax.ShapeDtypeStruct((G, 1), jnp.float32),
)


def kernel(x, edge_index, edge_attr, batch, W0, b0, g0, be0, W1, b1, g1, be1,
           W2, b2, W3, b3, fcW, fcb):
    ipad = jnp.zeros((EP - E,), jnp.int32)
    row = jnp.concatenate([edge_index[0].astype(jnp.int32), ipad]
                          ).reshape(NW, NCH, K)
    col = jnp.concatenate([edge_index[1].astype(jnp.int32), ipad]
                          ).reshape(NW, NCH, K)
    ewr = jnp.concatenate([edge_attr, jnp.zeros((EP - E,), jnp.float32)]
                          ).reshape(NW, NCH, K)
    batchp = jnp.concatenate(
        [batch.astype(jnp.int32), jnp.zeros((NP - N,), jnp.int32)])

    degP = _deg_kernel(col, ewr)
    h0p, dis = _tc_prep(degP, x, W0)
    P0 = _mp_kernel(h0p, row, col, ewr)
    h1p = _tc_combine_bn(P0, h0p, dis, b0, g0, be0, W1)
    P1 = _mp_kernel(h1p, row, col, ewr)
    h2p = _tc_combine_bn(P1, h1p, dis, b1, g1, be1, W2)
    P2 = _mp_kernel(h2p, row, col, ewr)
    h3p_self = _tc_combine_nobn(P2, h2p, dis, b2, jnp.zeros_like(g0),
                                jnp.zeros_like(be0), W3)
    P3 = _mp_kernel(h3p_self, row, col, ewr)
    h3pad, sm, cnt = _tc_final_a(P3, h3p_self, dis, b3, batch.astype(jnp.int32))
    maxP = _max_kernel(h3pad, batchp)
    return _tc_final_b(maxP, sm, cnt, fcW, fcb)
